# baseline (speedup 1.0000x reference)
import functools

import jax
import jax.numpy as jnp
from jax import lax
from jax.experimental import pallas as pl
from jax.experimental.pallas import tpu as pltpu
from jax.experimental.pallas import tpu_sc as plsc

N = 10000
D = 128
NC = 2
NS = 16
NW = NC * NS
LANES = 16
N_PAD = 10240
RPT = N_PAD // NS
CHUNK = 64
SUP = 8
E_PAD = 327680
EC = E_PAD // CHUNK
CPT = EC // NW
NSUP = CPT // SUP

BK = 1024
GRID = N_PAD // BK

_mesh = plsc.VectorSubcoreMesh(
    core_axis_name="c", subcore_axis_name="s", num_cores=NC, num_subcores=NS
)


@functools.partial(
    pl.kernel,
    out_type=jax.ShapeDtypeStruct((NC, N_PAD), jnp.float32),
    mesh=_mesh,
    scratch_types=[
        pltpu.VMEM((CPT, CHUNK), jnp.int32),
        pltpu.VMEM((CPT, CHUNK), jnp.float32),
        pltpu.VMEM((RPT,), jnp.float32),
        pltpu.VMEM_SHARED((N_PAD,), jnp.float32),
        pltpu.SemaphoreType.DMA,
    ],
)
def _deg_kernel(c_hbm, w_hbm, degp_hbm, cbuf, wbuf, zbuf, degs, ssem):
    cid = lax.axis_index("c")
    sid = lax.axis_index("s")
    wid = cid * NS + sid
    base = wid * CPT

    pltpu.sync_copy(c_hbm.at[pl.ds(base, CPT)], cbuf)
    pltpu.sync_copy(w_hbm.at[pl.ds(base, CPT)], wbuf)

    def _z(i, carry):
        zbuf[pl.ds(i * LANES, LANES)] = jnp.zeros((LANES,), jnp.float32)
        return carry

    lax.fori_loop(0, RPT // LANES, _z, None)
    pltpu.sync_copy(zbuf, degs.at[pl.ds(sid * RPT, RPT)])
    plsc.subcore_barrier()

    GRP = 16

    def _fire(j, carry):
        pltpu.async_copy(wbuf.at[j], degs.at[cbuf.at[j]], ssem, add=True)
        return carry

    def _drain(j, carry):
        pltpu.make_async_copy(wbuf.at[j], degs.at[cbuf.at[j]], ssem).wait()
        return carry

    for g in range(CPT // GRP):
        lax.fori_loop(g * GRP, (g + 1) * GRP, _fire, None)
        lax.fori_loop(g * GRP, (g + 1) * GRP, _drain, None)
    plsc.subcore_barrier()

    @pl.when(sid == 0)
    def _():
        pltpu.sync_copy(degs, degp_hbm.at[cid])


@functools.partial(
    pl.kernel,
    out_type=jax.ShapeDtypeStruct((NC, N_PAD, D), jnp.float32),
    mesh=_mesh,
    scratch_types=[
        pltpu.VMEM((2, SUP, CHUNK), jnp.int32),
        pltpu.VMEM((2, SUP, CHUNK), jnp.int32),
        pltpu.VMEM((2, SUP, CHUNK), jnp.float32),
        pltpu.VMEM((2, CHUNK, D), jnp.float32),
        pltpu.VMEM((2, CHUNK, D), jnp.float32),
        pltpu.VMEM_SHARED((N_PAD, D), jnp.float32),
        pltpu.SemaphoreType.DMA,
        pltpu.SemaphoreType.DMA,
        pltpu.SemaphoreType.DMA,
    ],
)
def _scatter_kernel(y_hbm, r_hbm, c_hbm, w_hbm, sp_hbm,
                    rbuf, cbuf, wbuf, gbuf, sbuf, acc, gsem, ssem, isem):
    cid = lax.axis_index("c")
    sid = lax.axis_index("s")
    wid = cid * NS + sid
    tb = wid * CPT

    pltpu.sync_copy(r_hbm.at[pl.ds(tb, SUP)], rbuf.at[0])
    pltpu.sync_copy(c_hbm.at[pl.ds(tb, SUP)], cbuf.at[0])
    pltpu.sync_copy(w_hbm.at[pl.ds(tb, SUP)], wbuf.at[0])
    pltpu.async_copy(y_hbm.at[rbuf.at[0, 0]], gbuf.at[0], gsem)
    pltpu.async_copy(y_hbm.at[rbuf.at[0, 1]], gbuf.at[1], gsem)

    def _zr(i, carry):
        for d in range(D // LANES):
            sbuf[0, i, pl.ds(d * LANES, LANES)] = jnp.zeros((LANES,), jnp.float32)
        return carry

    lax.fori_loop(0, CHUNK, _zr, None)
    for k in range(RPT // CHUNK):
        pltpu.sync_copy(sbuf.at[0], acc.at[pl.ds(sid * RPT + k * CHUNK, CHUNK)])
    plsc.subcore_barrier()

    @pl.loop(0, NSUP, step=2)
    def _sup(s0):
        for k in range(2):
            s = s0 + k
            kp = (k + 1) % 2
            nrow0 = tb + (s + 1) * SUP
            for j in range(SUP):
                b = j % 2
                pltpu.make_async_copy(
                    y_hbm.at[rbuf.at[k, j]], gbuf.at[b], gsem).wait()

                if j >= 2:
                    pltpu.make_async_copy(
                        sbuf.at[b], acc.at[cbuf.at[k, j - 2]], ssem).wait()
                else:
                    @pl.when(s >= 1)
                    def _():
                        pltpu.make_async_copy(
                            sbuf.at[b], acc.at[cbuf.at[kp, j + SUP - 2]],
                            ssem).wait()

                if j == 2:
                    @pl.when(s < NSUP - 1)
                    def _():
                        pltpu.async_copy(
                            r_hbm.at[pl.ds(nrow0, SUP)], rbuf.at[kp], isem)
                        pltpu.async_copy(
                            c_hbm.at[pl.ds(nrow0, SUP)], cbuf.at[kp], isem)
                        pltpu.async_copy(
                            w_hbm.at[pl.ds(nrow0, SUP)], wbuf.at[kp], isem)

                def _mul(gg, carry):
                    wv = wbuf[k, j, pl.ds(gg * LANES, LANES)]
                    bvs = tuple(jnp.full((LANES,), wv[l], jnp.float32)
                                for l in range(LANES))

                    def _dloop(d, bvs_):
                        sl = pl.ds(d * LANES, LANES)
                        for l in range(LANES):
                            e = gg * LANES + l
                            sbuf[b, e, sl] = gbuf[b, e, sl] * bvs_[l]
                        return bvs_

                    lax.fori_loop(0, D // LANES, _dloop, bvs)
                    return carry

                lax.fori_loop(0, CHUNK // LANES, _mul, None)
                pltpu.async_copy(sbuf.at[b], acc.at[cbuf.at[k, j]], ssem,
                                 add=True)

                if j == 5:
                    @pl.when(s < NSUP - 1)
                    def _():
                        pltpu.make_async_copy(
                            r_hbm.at[pl.ds(nrow0, SUP)], rbuf.at[kp],
                            isem).wait()
                        pltpu.make_async_copy(
                            c_hbm.at[pl.ds(nrow0, SUP)], cbuf.at[kp],
                            isem).wait()
                        pltpu.make_async_copy(
                            w_hbm.at[pl.ds(nrow0, SUP)], wbuf.at[kp],
                            isem).wait()

                if j < SUP - 2:
                    pltpu.async_copy(y_hbm.at[rbuf.at[k, j + 2]], gbuf.at[b],
                                     gsem)
                else:
                    @pl.when(s < NSUP - 1)
                    def _():
                        pltpu.async_copy(y_hbm.at[rbuf.at[kp, j - (SUP - 2)]],
                                         gbuf.at[b], gsem)

    for b in range(2):
        pltpu.make_async_copy(
            sbuf.at[b], acc.at[cbuf.at[1, SUP - 2 + b]], ssem).wait()
    plsc.subcore_barrier()

    for k in range(RPT // CHUNK):
        r0 = sid * RPT + k * CHUNK
        pltpu.sync_copy(acc.at[pl.ds(r0, CHUNK)], sp_hbm.at[cid, pl.ds(r0, CHUNK)])


def _tc_first(x_ref, w_ref, degp_ref, dis_ref, xw_ref, y_ref):
    deg = 1.0 + degp_ref[0] + degp_ref[1]
    dis = lax.rsqrt(deg)
    xw = jnp.dot(x_ref[...], w_ref[...], preferred_element_type=jnp.float32)
    dis_ref[...] = dis
    xw_ref[...] = xw
    y_ref[...] = xw * dis


def _stage_first(x_pad, W1, degp):
    return pl.pallas_call(
        _tc_first,
        grid=(GRID,),
        in_specs=[
            pl.BlockSpec((BK, D), lambda i: (i, 0)),
            pl.BlockSpec((D, D), lambda i: (0, 0)),
            pl.BlockSpec((NC, BK, 1), lambda i: (0, i, 0)),
        ],
        out_specs=[
            pl.BlockSpec((BK, 1), lambda i: (i, 0)),
            pl.BlockSpec((BK, D), lambda i: (i, 0)),
            pl.BlockSpec((BK, D), lambda i: (i, 0)),
        ],
        out_shape=[
            jax.ShapeDtypeStruct((N_PAD, 1), jnp.float32),
            jax.ShapeDtypeStruct((N_PAD, D), jnp.float32),
            jax.ShapeDtypeStruct((N_PAD, D), jnp.float32),
        ],
    )(x_pad, W1, degp)


def _tc_pre(sp_ref, xw_ref, dis_ref, b_ref, h_ref, ssum_ref, ssq_ref):
    i = pl.program_id(0)
    dis = dis_ref[...]
    h = (sp_ref[0] + sp_ref[1]) * dis + xw_ref[...] * (dis * dis) + b_ref[...]
    ridx = lax.broadcasted_iota(jnp.int32, (BK, 1), 0) + i * BK
    h = h * (ridx < N).astype(jnp.float32)
    h_ref[...] = h

    @pl.when(i == 0)
    def _():
        ssum_ref[...] = jnp.zeros_like(ssum_ref)
        ssq_ref[...] = jnp.zeros_like(ssq_ref)

    ssum_ref[...] += jnp.sum(h, axis=0, keepdims=True)
    ssq_ref[...] += jnp.sum(h * h, axis=0, keepdims=True)


def _stage_pre(sp, xw, dis, b):
    return pl.pallas_call(
        _tc_pre,
        grid=(GRID,),
        in_specs=[
            pl.BlockSpec((NC, BK, D), lambda i: (0, i, 0)),
            pl.BlockSpec((BK, D), lambda i: (i, 0)),
            pl.BlockSpec((BK, 1), lambda i: (i, 0)),
            pl.BlockSpec((1, D), lambda i: (0, 0)),
        ],
        out_specs=[
            pl.BlockSpec((BK, D), lambda i: (i, 0)),
            pl.BlockSpec((1, D), lambda i: (0, 0)),
            pl.BlockSpec((1, D), lambda i: (0, 0)),
        ],
        out_shape=[
            jax.ShapeDtypeStruct((N_PAD, D), jnp.float32),
            jax.ShapeDtypeStruct((1, D), jnp.float32),
            jax.ShapeDtypeStruct((1, D), jnp.float32),
        ],
    )(sp, xw, dis, b)


def _tc_post(h_ref, ssum_ref, ssq_ref, g_ref, be_ref, w_ref, dis_ref,
             xw2_ref, y2_ref):
    mean = ssum_ref[...] * (1.0 / N)
    var = ssq_ref[...] * (1.0 / N) - mean * mean
    inv = lax.rsqrt(var + 1e-5)
    h = (h_ref[...] - mean) * inv * g_ref[...] + be_ref[...]
    h = jnp.maximum(h, 0.0)
    xw2 = jnp.dot(h, w_ref[...], preferred_element_type=jnp.float32)
    xw2_ref[...] = xw2
    y2_ref[...] = xw2 * dis_ref[...]


def _stage_post(h, ssum, ssq, gamma, beta, Wn, dis):
    return pl.pallas_call(
        _tc_post,
        grid=(GRID,),
        in_specs=[
            pl.BlockSpec((BK, D), lambda i: (i, 0)),
            pl.BlockSpec((1, D), lambda i: (0, 0)),
            pl.BlockSpec((1, D), lambda i: (0, 0)),
            pl.BlockSpec((1, D), lambda i: (0, 0)),
            pl.BlockSpec((1, D), lambda i: (0, 0)),
            pl.BlockSpec((D, D), lambda i: (0, 0)),
            pl.BlockSpec((BK, 1), lambda i: (i, 0)),
        ],
        out_specs=[
            pl.BlockSpec((BK, D), lambda i: (i, 0)),
            pl.BlockSpec((BK, D), lambda i: (i, 0)),
        ],
        out_shape=[
            jax.ShapeDtypeStruct((N_PAD, D), jnp.float32),
            jax.ShapeDtypeStruct((N_PAD, D), jnp.float32),
        ],
    )(h, ssum, ssq, gamma, beta, Wn, dis)


def _tc_final(sp_ref, xw_ref, dis_ref, b_ref, out_ref):
    dis = dis_ref[...]
    h = (sp_ref[0] + sp_ref[1]) * dis + xw_ref[...] * (dis * dis) + b_ref[...]
    nrm = jnp.sqrt(jnp.sum(h * h, axis=1, keepdims=True))
    out_ref[...] = h / jnp.maximum(nrm, 1e-12)


def _stage_final(sp, xw, dis, b):
    return pl.pallas_call(
        _tc_final,
        grid=(GRID,),
        in_specs=[
            pl.BlockSpec((NC, BK, D), lambda i: (0, i, 0)),
            pl.BlockSpec((BK, D), lambda i: (i, 0)),
            pl.BlockSpec((BK, 1), lambda i: (i, 0)),
            pl.BlockSpec((1, D), lambda i: (0, 0)),
        ],
        out_specs=pl.BlockSpec((BK, D), lambda i: (i, 0)),
        out_shape=jax.ShapeDtypeStruct((N_PAD, D), jnp.float32),
    )(sp, xw, dis, b)


def kernel(x, edge_index, edge_attr, W1, b1, gamma1, beta1,
           W2, b2, gamma2, beta2, W3, b3):
    r = edge_index[0]
    c = edge_index[1]
    e = r.shape[0]
    pad_e = E_PAD - e
    fill = jnp.arange(pad_e, dtype=jnp.int32)
    r_p = jnp.concatenate([r, fill % N]).reshape(EC, CHUNK)
    c_p = jnp.concatenate([c, fill % N_PAD]).reshape(EC, CHUNK)
    w_p = jnp.concatenate(
        [edge_attr, jnp.zeros((pad_e,), jnp.float32)]).reshape(EC, CHUNK)
    x_pad = jnp.pad(x, ((0, N_PAD - N), (0, 0)))

    degp = _deg_kernel(c_p, w_p).reshape(NC, N_PAD, 1)
    dis, xw1, y1 = _stage_first(x_pad, W1, degp)

    sp1 = _scatter_kernel(y1, r_p, c_p, w_p)
    h1, s1, q1 = _stage_pre(sp1, xw1, dis, b1.reshape(1, D))
    xw2, y2 = _stage_post(h1, s1, q1, gamma1.reshape(1, D),
                          beta1.reshape(1, D), W2, dis)

    sp2 = _scatter_kernel(y2, r_p, c_p, w_p)
    h2, s2, q2 = _stage_pre(sp2, xw2, dis, b2.reshape(1, D))
    xw3, y3 = _stage_post(h2, s2, q2, gamma2.reshape(1, D),
                          beta2.reshape(1, D), W3, dis)

    sp3 = _scatter_kernel(y3, r_p, c_p, w_p)
    out = _stage_final(sp3, xw3, dis, b3.reshape(1, D))
    return out[:N]


# speedup vs baseline: 1.5722x; 1.5722x over previous
import functools

import jax
import jax.numpy as jnp
from jax import lax
from jax.experimental import pallas as pl
from jax.experimental.pallas import tpu as pltpu
from jax.experimental.pallas import tpu_sc as plsc

N = 10000
D = 128
NC = 2
NS = 16
NW = NC * NS
LANES = 16
N_PAD = 10240
RPT = N_PAD // NS
CHUNK = 128
SUP = 8
E_PAD = 327680
EC = E_PAD // CHUNK
CPT = EC // NW
NSUP = CPT // SUP

BK = 1024
GRID = N_PAD // BK

_mesh = plsc.VectorSubcoreMesh(
    core_axis_name="c", subcore_axis_name="s", num_cores=NC, num_subcores=NS
)


@functools.partial(
    pl.kernel,
    out_type=jax.ShapeDtypeStruct((NC, N_PAD), jnp.float32),
    mesh=_mesh,
    scratch_types=[
        pltpu.VMEM((CPT, CHUNK), jnp.int32),
        pltpu.VMEM((CPT, CHUNK), jnp.float32),
        pltpu.VMEM((RPT,), jnp.float32),
        pltpu.VMEM_SHARED((N_PAD,), jnp.float32),
        pltpu.SemaphoreType.DMA,
    ],
)
def _deg_kernel(c_hbm, w_hbm, degp_hbm, cbuf, wbuf, zbuf, degs, ssem):
    cid = lax.axis_index("c")
    sid = lax.axis_index("s")
    wid = cid * NS + sid
    base = wid * CPT

    pltpu.sync_copy(c_hbm.at[pl.ds(base, CPT)], cbuf)
    pltpu.sync_copy(w_hbm.at[pl.ds(base, CPT)], wbuf)

    def _z(i, carry):
        zbuf[pl.ds(i * LANES, LANES)] = jnp.zeros((LANES,), jnp.float32)
        return carry

    lax.fori_loop(0, RPT // LANES, _z, None)
    pltpu.sync_copy(zbuf, degs.at[pl.ds(sid * RPT, RPT)])
    plsc.subcore_barrier()

    GRP = 16

    def _fire(j, carry):
        pltpu.async_copy(wbuf.at[j], degs.at[cbuf.at[j]], ssem, add=True)
        return carry

    def _drain(j, carry):
        pltpu.make_async_copy(wbuf.at[j], degs.at[cbuf.at[j]], ssem).wait()
        return carry

    for g in range(CPT // GRP):
        lax.fori_loop(g * GRP, (g + 1) * GRP, _fire, None)
        lax.fori_loop(g * GRP, (g + 1) * GRP, _drain, None)
    plsc.subcore_barrier()

    @pl.when(sid == 0)
    def _():
        pltpu.sync_copy(degs, degp_hbm.at[cid])


@functools.partial(
    pl.kernel,
    out_type=jax.ShapeDtypeStruct((NC, N_PAD, D), jnp.float32),
    mesh=_mesh,
    scratch_types=[
        pltpu.VMEM((SUP, CHUNK), jnp.int32),
        pltpu.VMEM((SUP, CHUNK), jnp.int32),
        pltpu.VMEM((SUP, CHUNK), jnp.float32),
        pltpu.VMEM((CHUNK, D), jnp.float32),
        pltpu.VMEM((CHUNK, D), jnp.float32),
        pltpu.VMEM_SHARED((N_PAD, D), jnp.float32),
        pltpu.SemaphoreType.DMA,
    ],
)
def _scatter_kernel(y_hbm, r_hbm, c_hbm, w_hbm, sp_hbm,
                    rbuf, cbuf, wbuf, gbuf, sbuf, acc, ssem):
    cid = lax.axis_index("c")
    sid = lax.axis_index("s")
    wid = cid * NS + sid
    tb = wid * CPT

    def _zr(i, carry):
        for d in range(D // LANES):
            sbuf[i, pl.ds(d * LANES, LANES)] = jnp.zeros((LANES,), jnp.float32)
        return carry

    lax.fori_loop(0, CHUNK, _zr, None)
    for k in range(RPT // CHUNK):
        pltpu.sync_copy(sbuf, acc.at[pl.ds(sid * RPT + k * CHUNK, CHUNK)])
    plsc.subcore_barrier()

    def _sup(s, carry):
        @pl.when(s >= 1)
        def _():
            pltpu.make_async_copy(
                sbuf, acc.at[cbuf.at[SUP - 1]], ssem).wait()

        row0 = tb + s * SUP
        pltpu.sync_copy(r_hbm.at[pl.ds(row0, SUP)], rbuf)
        pltpu.sync_copy(c_hbm.at[pl.ds(row0, SUP)], cbuf)
        pltpu.sync_copy(w_hbm.at[pl.ds(row0, SUP)], wbuf)

        for j in range(SUP):
            pltpu.sync_copy(y_hbm.at[rbuf.at[j]], gbuf)

            if j >= 1:
                pltpu.make_async_copy(
                    sbuf, acc.at[cbuf.at[j - 1]], ssem).wait()

            def _mul(gg, carry2):
                wv = wbuf[j, pl.ds(gg * LANES, LANES)]
                for l in range(LANES):
                    bv = jnp.full((LANES,), wv[l], jnp.float32)
                    for d in range(D // LANES):
                        sl = pl.ds(d * LANES, LANES)
                        sbuf[gg * LANES + l, sl] = gbuf[gg * LANES + l, sl] * bv
                return carry2

            lax.fori_loop(0, CHUNK // LANES, _mul, None)
            pltpu.async_copy(sbuf, acc.at[cbuf.at[j]], ssem, add=True)
        return carry

    lax.fori_loop(0, NSUP, _sup, None)
    pltpu.make_async_copy(sbuf, acc.at[cbuf.at[SUP - 1]], ssem).wait()
    plsc.subcore_barrier()

    for k in range(RPT // CHUNK):
        r0 = sid * RPT + k * CHUNK
        pltpu.sync_copy(acc.at[pl.ds(r0, CHUNK)], sp_hbm.at[cid, pl.ds(r0, CHUNK)])


def _tc_first(x_ref, w_ref, degp_ref, dis_ref, xw_ref, y_ref):
    deg = 1.0 + degp_ref[0] + degp_ref[1]
    dis = lax.rsqrt(deg)
    xw = jnp.dot(x_ref[...], w_ref[...], preferred_element_type=jnp.float32)
    dis_ref[...] = dis
    xw_ref[...] = xw
    y_ref[...] = xw * dis


def _stage_first(x_pad, W1, degp):
    return pl.pallas_call(
        _tc_first,
        grid=(GRID,),
        in_specs=[
            pl.BlockSpec((BK, D), lambda i: (i, 0)),
            pl.BlockSpec((D, D), lambda i: (0, 0)),
            pl.BlockSpec((NC, BK, 1), lambda i: (0, i, 0)),
        ],
        out_specs=[
            pl.BlockSpec((BK, 1), lambda i: (i, 0)),
            pl.BlockSpec((BK, D), lambda i: (i, 0)),
            pl.BlockSpec((BK, D), lambda i: (i, 0)),
        ],
        out_shape=[
            jax.ShapeDtypeStruct((N_PAD, 1), jnp.float32),
            jax.ShapeDtypeStruct((N_PAD, D), jnp.float32),
            jax.ShapeDtypeStruct((N_PAD, D), jnp.float32),
        ],
    )(x_pad, W1, degp)


def _tc_pre(sp_ref, xw_ref, dis_ref, b_ref, h_ref, ssum_ref, ssq_ref):
    i = pl.program_id(0)
    dis = dis_ref[...]
    h = (sp_ref[0] + sp_ref[1]) * dis + xw_ref[...] * (dis * dis) + b_ref[...]
    ridx = lax.broadcasted_iota(jnp.int32, (BK, 1), 0) + i * BK
    h = h * (ridx < N).astype(jnp.float32)
    h_ref[...] = h

    @pl.when(i == 0)
    def _():
        ssum_ref[...] = jnp.zeros_like(ssum_ref)
        ssq_ref[...] = jnp.zeros_like(ssq_ref)

    ssum_ref[...] += jnp.sum(h, axis=0, keepdims=True)
    ssq_ref[...] += jnp.sum(h * h, axis=0, keepdims=True)


def _stage_pre(sp, xw, dis, b):
    return pl.pallas_call(
        _tc_pre,
        grid=(GRID,),
        in_specs=[
            pl.BlockSpec((NC, BK, D), lambda i: (0, i, 0)),
            pl.BlockSpec((BK, D), lambda i: (i, 0)),
            pl.BlockSpec((BK, 1), lambda i: (i, 0)),
            pl.BlockSpec((1, D), lambda i: (0, 0)),
        ],
        out_specs=[
            pl.BlockSpec((BK, D), lambda i: (i, 0)),
            pl.BlockSpec((1, D), lambda i: (0, 0)),
            pl.BlockSpec((1, D), lambda i: (0, 0)),
        ],
        out_shape=[
            jax.ShapeDtypeStruct((N_PAD, D), jnp.float32),
            jax.ShapeDtypeStruct((1, D), jnp.float32),
            jax.ShapeDtypeStruct((1, D), jnp.float32),
        ],
    )(sp, xw, dis, b)


def _tc_post(h_ref, ssum_ref, ssq_ref, g_ref, be_ref, w_ref, dis_ref,
             xw2_ref, y2_ref):
    mean = ssum_ref[...] * (1.0 / N)
    var = ssq_ref[...] * (1.0 / N) - mean * mean
    inv = lax.rsqrt(var + 1e-5)
    h = (h_ref[...] - mean) * inv * g_ref[...] + be_ref[...]
    h = jnp.maximum(h, 0.0)
    xw2 = jnp.dot(h, w_ref[...], preferred_element_type=jnp.float32)
    xw2_ref[...] = xw2
    y2_ref[...] = xw2 * dis_ref[...]


def _stage_post(h, ssum, ssq, gamma, beta, Wn, dis):
    return pl.pallas_call(
        _tc_post,
        grid=(GRID,),
        in_specs=[
            pl.BlockSpec((BK, D), lambda i: (i, 0)),
            pl.BlockSpec((1, D), lambda i: (0, 0)),
            pl.BlockSpec((1, D), lambda i: (0, 0)),
            pl.BlockSpec((1, D), lambda i: (0, 0)),
            pl.BlockSpec((1, D), lambda i: (0, 0)),
            pl.BlockSpec((D, D), lambda i: (0, 0)),
            pl.BlockSpec((BK, 1), lambda i: (i, 0)),
        ],
        out_specs=[
            pl.BlockSpec((BK, D), lambda i: (i, 0)),
            pl.BlockSpec((BK, D), lambda i: (i, 0)),
        ],
        out_shape=[
            jax.ShapeDtypeStruct((N_PAD, D), jnp.float32),
            jax.ShapeDtypeStruct((N_PAD, D), jnp.float32),
        ],
    )(h, ssum, ssq, gamma, beta, Wn, dis)


def _tc_final(sp_ref, xw_ref, dis_ref, b_ref, out_ref):
    dis = dis_ref[...]
    h = (sp_ref[0] + sp_ref[1]) * dis + xw_ref[...] * (dis * dis) + b_ref[...]
    nrm = jnp.sqrt(jnp.sum(h * h, axis=1, keepdims=True))
    out_ref[...] = h / jnp.maximum(nrm, 1e-12)


def _stage_final(sp, xw, dis, b):
    return pl.pallas_call(
        _tc_final,
        grid=(GRID,),
        in_specs=[
            pl.BlockSpec((NC, BK, D), lambda i: (0, i, 0)),
            pl.BlockSpec((BK, D), lambda i: (i, 0)),
            pl.BlockSpec((BK, 1), lambda i: (i, 0)),
            pl.BlockSpec((1, D), lambda i: (0, 0)),
        ],
        out_specs=pl.BlockSpec((BK, D), lambda i: (i, 0)),
        out_shape=jax.ShapeDtypeStruct((N_PAD, D), jnp.float32),
    )(sp, xw, dis, b)


def kernel(x, edge_index, edge_attr, W1, b1, gamma1, beta1,
           W2, b2, gamma2, beta2, W3, b3):
    r = edge_index[0]
    c = edge_index[1]
    e = r.shape[0]
    pad_e = E_PAD - e
    fill = jnp.arange(pad_e, dtype=jnp.int32)
    r_p = jnp.concatenate([r, fill % N]).reshape(EC, CHUNK)
    c_p = jnp.concatenate([c, fill % N_PAD]).reshape(EC, CHUNK)
    w_p = jnp.concatenate(
        [edge_attr, jnp.zeros((pad_e,), jnp.float32)]).reshape(EC, CHUNK)
    x_pad = jnp.pad(x, ((0, N_PAD - N), (0, 0)))

    degp = _deg_kernel(c_p, w_p).reshape(NC, N_PAD, 1)
    dis, xw1, y1 = _stage_first(x_pad, W1, degp)

    sp1 = _scatter_kernel(y1, r_p, c_p, w_p)
    h1, s1, q1 = _stage_pre(sp1, xw1, dis, b1.reshape(1, D))
    xw2, y2 = _stage_post(h1, s1, q1, gamma1.reshape(1, D),
                          beta1.reshape(1, D), W2, dis)

    sp2 = _scatter_kernel(y2, r_p, c_p, w_p)
    h2, s2, q2 = _stage_pre(sp2, xw2, dis, b2.reshape(1, D))
    xw3, y3 = _stage_post(h2, s2, q2, gamma2.reshape(1, D),
                          beta2.reshape(1, D), W3, dis)

    sp3 = _scatter_kernel(y3, r_p, c_p, w_p)
    out = _stage_final(sp3, xw3, dis, b3.reshape(1, D))
    return out[:N]


# speedup vs baseline: 1.6102x; 1.0242x over previous
import functools

import jax
import jax.numpy as jnp
from jax import lax
from jax.experimental import pallas as pl
from jax.experimental.pallas import tpu as pltpu
from jax.experimental.pallas import tpu_sc as plsc

N = 10000
D = 128
NC = 2
NS = 16
NW = NC * NS
LANES = 16
N_PAD = 10240
RPT = N_PAD // NS
CHUNK = 128
SUP = 8
E_PAD = 327680
EC = E_PAD // CHUNK
CPT = EC // NW
NSUP = CPT // SUP

BK = 1024
GRID = N_PAD // BK

_mesh = plsc.VectorSubcoreMesh(
    core_axis_name="c", subcore_axis_name="s", num_cores=NC, num_subcores=NS
)


@functools.partial(
    pl.kernel,
    out_type=jax.ShapeDtypeStruct((NC, N_PAD), jnp.float32),
    mesh=_mesh,
    scratch_types=[
        pltpu.VMEM((CPT, CHUNK), jnp.int32),
        pltpu.VMEM((CPT, CHUNK), jnp.float32),
        pltpu.VMEM((RPT,), jnp.float32),
        pltpu.VMEM_SHARED((N_PAD,), jnp.float32),
        pltpu.SemaphoreType.DMA,
    ],
)
def _deg_kernel(c_hbm, w_hbm, degp_hbm, cbuf, wbuf, zbuf, degs, ssem):
    cid = lax.axis_index("c")
    sid = lax.axis_index("s")
    wid = cid * NS + sid
    base = wid * CPT

    pltpu.sync_copy(c_hbm.at[pl.ds(base, CPT)], cbuf)
    pltpu.sync_copy(w_hbm.at[pl.ds(base, CPT)], wbuf)

    def _z(i, carry):
        zbuf[pl.ds(i * LANES, LANES)] = jnp.zeros((LANES,), jnp.float32)
        return carry

    lax.fori_loop(0, RPT // LANES, _z, None)
    pltpu.sync_copy(zbuf, degs.at[pl.ds(sid * RPT, RPT)])
    plsc.subcore_barrier()

    GRP = 16

    def _fire(j, carry):
        pltpu.async_copy(wbuf.at[j], degs.at[cbuf.at[j]], ssem, add=True)
        return carry

    def _drain(j, carry):
        pltpu.make_async_copy(wbuf.at[j], degs.at[cbuf.at[j]], ssem).wait()
        return carry

    for g in range(CPT // GRP):
        lax.fori_loop(g * GRP, (g + 1) * GRP, _fire, None)
        lax.fori_loop(g * GRP, (g + 1) * GRP, _drain, None)
    plsc.subcore_barrier()

    @pl.when(sid == 0)
    def _():
        pltpu.sync_copy(degs, degp_hbm.at[cid])


@functools.partial(
    pl.kernel,
    out_type=jax.ShapeDtypeStruct((NC, N_PAD, D), jnp.float32),
    mesh=_mesh,
    scratch_types=[
        pltpu.VMEM((SUP, CHUNK), jnp.int32),
        pltpu.VMEM((SUP, CHUNK), jnp.int32),
        pltpu.VMEM((SUP, CHUNK), jnp.float32),
        pltpu.VMEM((CHUNK, D), jnp.float32),
        pltpu.VMEM((CHUNK, D), jnp.float32),
        pltpu.VMEM_SHARED((N_PAD, D), jnp.float32),
        pltpu.SemaphoreType.DMA,
    ],
)
def _scatter_kernel(y_hbm, r_hbm, c_hbm, w_hbm, sp_hbm,
                    rbuf, cbuf, wbuf, gbuf, sbuf, acc, ssem):
    cid = lax.axis_index("c")
    sid = lax.axis_index("s")
    wid = cid * NS + sid
    tb = wid * CPT

    def _zr(i, carry):
        for d in range(D // LANES):
            sbuf[i, pl.ds(d * LANES, LANES)] = jnp.zeros((LANES,), jnp.float32)
        return carry

    lax.fori_loop(0, CHUNK, _zr, None)
    for k in range(RPT // CHUNK):
        pltpu.sync_copy(sbuf, acc.at[pl.ds(sid * RPT + k * CHUNK, CHUNK)])
    plsc.subcore_barrier()

    def _sup(s, carry):
        row0 = tb + s * SUP
        pltpu.sync_copy(r_hbm.at[pl.ds(row0, SUP)], rbuf)
        pltpu.sync_copy(c_hbm.at[pl.ds(row0, SUP)], cbuf)
        pltpu.sync_copy(w_hbm.at[pl.ds(row0, SUP)], wbuf)

        for j in range(SUP):
            pltpu.sync_copy(y_hbm.at[rbuf.at[j]], gbuf)

            def _mul(gg, carry2):
                wv = wbuf[j, pl.ds(gg * LANES, LANES)]
                for l in range(LANES):
                    bv = jnp.full((LANES,), wv[l], jnp.float32)
                    for d in range(D // LANES):
                        sl = pl.ds(d * LANES, LANES)
                        sbuf[gg * LANES + l, sl] = gbuf[gg * LANES + l, sl] * bv
                return carry2

            lax.fori_loop(0, CHUNK // LANES, _mul, None)
        return carry

    lax.fori_loop(0, NSUP, _sup, None)
    plsc.subcore_barrier()

    for k in range(RPT // CHUNK):
        r0 = sid * RPT + k * CHUNK
        pltpu.sync_copy(acc.at[pl.ds(r0, CHUNK)], sp_hbm.at[cid, pl.ds(r0, CHUNK)])


def _tc_first(x_ref, w_ref, degp_ref, dis_ref, xw_ref, y_ref):
    deg = 1.0 + degp_ref[0] + degp_ref[1]
    dis = lax.rsqrt(deg)
    xw = jnp.dot(x_ref[...], w_ref[...], preferred_element_type=jnp.float32)
    dis_ref[...] = dis
    xw_ref[...] = xw
    y_ref[...] = xw * dis


def _stage_first(x_pad, W1, degp):
    return pl.pallas_call(
        _tc_first,
        grid=(GRID,),
        in_specs=[
            pl.BlockSpec((BK, D), lambda i: (i, 0)),
            pl.BlockSpec((D, D), lambda i: (0, 0)),
            pl.BlockSpec((NC, BK, 1), lambda i: (0, i, 0)),
        ],
        out_specs=[
            pl.BlockSpec((BK, 1), lambda i: (i, 0)),
            pl.BlockSpec((BK, D), lambda i: (i, 0)),
            pl.BlockSpec((BK, D), lambda i: (i, 0)),
        ],
        out_shape=[
            jax.ShapeDtypeStruct((N_PAD, 1), jnp.float32),
            jax.ShapeDtypeStruct((N_PAD, D), jnp.float32),
            jax.ShapeDtypeStruct((N_PAD, D), jnp.float32),
        ],
    )(x_pad, W1, degp)


def _tc_pre(sp_ref, xw_ref, dis_ref, b_ref, h_ref, ssum_ref, ssq_ref):
    i = pl.program_id(0)
    dis = dis_ref[...]
    h = (sp_ref[0] + sp_ref[1]) * dis + xw_ref[...] * (dis * dis) + b_ref[...]
    ridx = lax.broadcasted_iota(jnp.int32, (BK, 1), 0) + i * BK
    h = h * (ridx < N).astype(jnp.float32)
    h_ref[...] = h

    @pl.when(i == 0)
    def _():
        ssum_ref[...] = jnp.zeros_like(ssum_ref)
        ssq_ref[...] = jnp.zeros_like(ssq_ref)

    ssum_ref[...] += jnp.sum(h, axis=0, keepdims=True)
    ssq_ref[...] += jnp.sum(h * h, axis=0, keepdims=True)


def _stage_pre(sp, xw, dis, b):
    return pl.pallas_call(
        _tc_pre,
        grid=(GRID,),
        in_specs=[
            pl.BlockSpec((NC, BK, D), lambda i: (0, i, 0)),
            pl.BlockSpec((BK, D), lambda i: (i, 0)),
            pl.BlockSpec((BK, 1), lambda i: (i, 0)),
            pl.BlockSpec((1, D), lambda i: (0, 0)),
        ],
        out_specs=[
            pl.BlockSpec((BK, D), lambda i: (i, 0)),
            pl.BlockSpec((1, D), lambda i: (0, 0)),
            pl.BlockSpec((1, D), lambda i: (0, 0)),
        ],
        out_shape=[
            jax.ShapeDtypeStruct((N_PAD, D), jnp.float32),
            jax.ShapeDtypeStruct((1, D), jnp.float32),
            jax.ShapeDtypeStruct((1, D), jnp.float32),
        ],
    )(sp, xw, dis, b)


def _tc_post(h_ref, ssum_ref, ssq_ref, g_ref, be_ref, w_ref, dis_ref,
             xw2_ref, y2_ref):
    mean = ssum_ref[...] * (1.0 / N)
    var = ssq_ref[...] * (1.0 / N) - mean * mean
    inv = lax.rsqrt(var + 1e-5)
    h = (h_ref[...] - mean) * inv * g_ref[...] + be_ref[...]
    h = jnp.maximum(h, 0.0)
    xw2 = jnp.dot(h, w_ref[...], preferred_element_type=jnp.float32)
    xw2_ref[...] = xw2
    y2_ref[...] = xw2 * dis_ref[...]


def _stage_post(h, ssum, ssq, gamma, beta, Wn, dis):
    return pl.pallas_call(
        _tc_post,
        grid=(GRID,),
        in_specs=[
            pl.BlockSpec((BK, D), lambda i: (i, 0)),
            pl.BlockSpec((1, D), lambda i: (0, 0)),
            pl.BlockSpec((1, D), lambda i: (0, 0)),
            pl.BlockSpec((1, D), lambda i: (0, 0)),
            pl.BlockSpec((1, D), lambda i: (0, 0)),
            pl.BlockSpec((D, D), lambda i: (0, 0)),
            pl.BlockSpec((BK, 1), lambda i: (i, 0)),
        ],
        out_specs=[
            pl.BlockSpec((BK, D), lambda i: (i, 0)),
            pl.BlockSpec((BK, D), lambda i: (i, 0)),
        ],
        out_shape=[
            jax.ShapeDtypeStruct((N_PAD, D), jnp.float32),
            jax.ShapeDtypeStruct((N_PAD, D), jnp.float32),
        ],
    )(h, ssum, ssq, gamma, beta, Wn, dis)


def _tc_final(sp_ref, xw_ref, dis_ref, b_ref, out_ref):
    dis = dis_ref[...]
    h = (sp_ref[0] + sp_ref[1]) * dis + xw_ref[...] * (dis * dis) + b_ref[...]
    nrm = jnp.sqrt(jnp.sum(h * h, axis=1, keepdims=True))
    out_ref[...] = h / jnp.maximum(nrm, 1e-12)


def _stage_final(sp, xw, dis, b):
    return pl.pallas_call(
        _tc_final,
        grid=(GRID,),
        in_specs=[
            pl.BlockSpec((NC, BK, D), lambda i: (0, i, 0)),
            pl.BlockSpec((BK, D), lambda i: (i, 0)),
            pl.BlockSpec((BK, 1), lambda i: (i, 0)),
            pl.BlockSpec((1, D), lambda i: (0, 0)),
        ],
        out_specs=pl.BlockSpec((BK, D), lambda i: (i, 0)),
        out_shape=jax.ShapeDtypeStruct((N_PAD, D), jnp.float32),
    )(sp, xw, dis, b)


def kernel(x, edge_index, edge_attr, W1, b1, gamma1, beta1,
           W2, b2, gamma2, beta2, W3, b3):
    r = edge_index[0]
    c = edge_index[1]
    e = r.shape[0]
    pad_e = E_PAD - e
    fill = jnp.arange(pad_e, dtype=jnp.int32)
    r_p = jnp.concatenate([r, fill % N]).reshape(EC, CHUNK)
    c_p = jnp.concatenate([c, fill % N_PAD]).reshape(EC, CHUNK)
    w_p = jnp.concatenate(
        [edge_attr, jnp.zeros((pad_e,), jnp.float32)]).reshape(EC, CHUNK)
    x_pad = jnp.pad(x, ((0, N_PAD - N), (0, 0)))

    degp = _deg_kernel(c_p, w_p).reshape(NC, N_PAD, 1)
    dis, xw1, y1 = _stage_first(x_pad, W1, degp)

    sp1 = _scatter_kernel(y1, r_p, c_p, w_p)
    h1, s1, q1 = _stage_pre(sp1, xw1, dis, b1.reshape(1, D))
    xw2, y2 = _stage_post(h1, s1, q1, gamma1.reshape(1, D),
                          beta1.reshape(1, D), W2, dis)

    sp2 = _scatter_kernel(y2, r_p, c_p, w_p)
    h2, s2, q2 = _stage_pre(sp2, xw2, dis, b2.reshape(1, D))
    xw3, y3 = _stage_post(h2, s2, q2, gamma2.reshape(1, D),
                          beta2.reshape(1, D), W3, dis)

    sp3 = _scatter_kernel(y3, r_p, c_p, w_p)
    out = _stage_final(sp3, xw3, dis, b3.reshape(1, D))
    return out[:N]


# speedup vs baseline: 2.6188x; 1.6263x over previous
import functools

import jax
import jax.numpy as jnp
from jax import lax
from jax.experimental import pallas as pl
from jax.experimental.pallas import tpu as pltpu
from jax.experimental.pallas import tpu_sc as plsc

N = 10000
D = 128
NC = 2
NS = 16
NW = NC * NS
LANES = 16
N_PAD = 10240
RPT = N_PAD // NS
CHUNK = 128
SUP = 8
E_PAD = 327680
EC = E_PAD // CHUNK
CPT = EC // NW
NSUP = CPT // SUP

BK = 1024
GRID = N_PAD // BK

_mesh = plsc.VectorSubcoreMesh(
    core_axis_name="c", subcore_axis_name="s", num_cores=NC, num_subcores=NS
)


@functools.partial(
    pl.kernel,
    out_type=jax.ShapeDtypeStruct((NC, N_PAD), jnp.float32),
    mesh=_mesh,
    scratch_types=[
        pltpu.VMEM((CPT, CHUNK), jnp.int32),
        pltpu.VMEM((CPT, CHUNK), jnp.float32),
        pltpu.VMEM((RPT,), jnp.float32),
        pltpu.VMEM_SHARED((N_PAD,), jnp.float32),
        pltpu.SemaphoreType.DMA,
    ],
)
def _deg_kernel(c_hbm, w_hbm, degp_hbm, cbuf, wbuf, zbuf, degs, ssem):
    cid = lax.axis_index("c")
    sid = lax.axis_index("s")
    wid = cid * NS + sid
    base = wid * CPT

    pltpu.sync_copy(c_hbm.at[pl.ds(base, CPT)], cbuf)
    pltpu.sync_copy(w_hbm.at[pl.ds(base, CPT)], wbuf)

    def _z(i, carry):
        zbuf[pl.ds(i * LANES, LANES)] = jnp.zeros((LANES,), jnp.float32)
        return carry

    lax.fori_loop(0, RPT // LANES, _z, None)
    pltpu.sync_copy(zbuf, degs.at[pl.ds(sid * RPT, RPT)])
    plsc.subcore_barrier()

    GRP = 16

    def _fire(j, carry):
        pltpu.async_copy(wbuf.at[j], degs.at[cbuf.at[j]], ssem, add=True)
        return carry

    def _drain(j, carry):
        pltpu.make_async_copy(wbuf.at[j], degs.at[cbuf.at[j]], ssem).wait()
        return carry

    for g in range(CPT // GRP):
        lax.fori_loop(g * GRP, (g + 1) * GRP, _fire, None)
        lax.fori_loop(g * GRP, (g + 1) * GRP, _drain, None)
    plsc.subcore_barrier()

    @pl.when(sid == 0)
    def _():
        pltpu.sync_copy(degs, degp_hbm.at[cid])


@functools.partial(
    pl.kernel,
    out_type=jax.ShapeDtypeStruct((NC, N_PAD, D), jnp.float32),
    mesh=_mesh,
    scratch_types=[
        pltpu.VMEM((SUP, CHUNK), jnp.int32),
        pltpu.VMEM((SUP, CHUNK), jnp.int32),
        pltpu.VMEM((SUP, CHUNK), jnp.float32),
        pltpu.VMEM((CHUNK, D), jnp.float32),
        pltpu.VMEM((CHUNK, D), jnp.float32),
        pltpu.VMEM_SHARED((N_PAD, D), jnp.float32),
        pltpu.SemaphoreType.DMA,
    ],
)
def _scatter_kernel(y_hbm, r_hbm, c_hbm, w_hbm, sp_hbm,
                    rbuf, cbuf, wbuf, gbuf, sbuf, acc, ssem):
    cid = lax.axis_index("c")
    sid = lax.axis_index("s")
    wid = cid * NS + sid
    tb = wid * CPT

    def _zr(i, carry):
        for d in range(D // LANES):
            sbuf[i, pl.ds(d * LANES, LANES)] = jnp.zeros((LANES,), jnp.float32)
        return carry

    lax.fori_loop(0, CHUNK, _zr, None)
    for k in range(RPT // CHUNK):
        pltpu.sync_copy(sbuf, acc.at[pl.ds(sid * RPT + k * CHUNK, CHUNK)])
    plsc.subcore_barrier()

    def _sup(s, carry):
        row0 = tb + s * SUP
        pltpu.sync_copy(r_hbm.at[pl.ds(row0, SUP)], rbuf)
        pltpu.sync_copy(c_hbm.at[pl.ds(row0, SUP)], cbuf)
        pltpu.sync_copy(w_hbm.at[pl.ds(row0, SUP)], wbuf)

        for j in range(SUP):
            pltpu.sync_copy(y_hbm.at[rbuf.at[j]], gbuf)

            def _mul(gg, carry2):
                wv = wbuf[j, pl.ds(gg * LANES, LANES)]
                for l in range(LANES):
                    bv = jnp.full((LANES,), wv[l], jnp.float32)
                    for d in range(D // LANES):
                        sl = pl.ds(d * LANES, LANES)
                        sbuf[gg * LANES + l, sl] = gbuf[gg * LANES + l, sl] * bv
                return carry2

        return carry

    lax.fori_loop(0, NSUP, _sup, None)
    plsc.subcore_barrier()

    for k in range(RPT // CHUNK):
        r0 = sid * RPT + k * CHUNK
        pltpu.sync_copy(acc.at[pl.ds(r0, CHUNK)], sp_hbm.at[cid, pl.ds(r0, CHUNK)])


def _tc_first(x_ref, w_ref, degp_ref, dis_ref, xw_ref, y_ref):
    deg = 1.0 + degp_ref[0] + degp_ref[1]
    dis = lax.rsqrt(deg)
    xw = jnp.dot(x_ref[...], w_ref[...], preferred_element_type=jnp.float32)
    dis_ref[...] = dis
    xw_ref[...] = xw
    y_ref[...] = xw * dis


def _stage_first(x_pad, W1, degp):
    return pl.pallas_call(
        _tc_first,
        grid=(GRID,),
        in_specs=[
            pl.BlockSpec((BK, D), lambda i: (i, 0)),
            pl.BlockSpec((D, D), lambda i: (0, 0)),
            pl.BlockSpec((NC, BK, 1), lambda i: (0, i, 0)),
        ],
        out_specs=[
            pl.BlockSpec((BK, 1), lambda i: (i, 0)),
            pl.BlockSpec((BK, D), lambda i: (i, 0)),
            pl.BlockSpec((BK, D), lambda i: (i, 0)),
        ],
        out_shape=[
            jax.ShapeDtypeStruct((N_PAD, 1), jnp.float32),
            jax.ShapeDtypeStruct((N_PAD, D), jnp.float32),
            jax.ShapeDtypeStruct((N_PAD, D), jnp.float32),
        ],
    )(x_pad, W1, degp)


def _tc_pre(sp_ref, xw_ref, dis_ref, b_ref, h_ref, ssum_ref, ssq_ref):
    i = pl.program_id(0)
    dis = dis_ref[...]
    h = (sp_ref[0] + sp_ref[1]) * dis + xw_ref[...] * (dis * dis) + b_ref[...]
    ridx = lax.broadcasted_iota(jnp.int32, (BK, 1), 0) + i * BK
    h = h * (ridx < N).astype(jnp.float32)
    h_ref[...] = h

    @pl.when(i == 0)
    def _():
        ssum_ref[...] = jnp.zeros_like(ssum_ref)
        ssq_ref[...] = jnp.zeros_like(ssq_ref)

    ssum_ref[...] += jnp.sum(h, axis=0, keepdims=True)
    ssq_ref[...] += jnp.sum(h * h, axis=0, keepdims=True)


def _stage_pre(sp, xw, dis, b):
    return pl.pallas_call(
        _tc_pre,
        grid=(GRID,),
        in_specs=[
            pl.BlockSpec((NC, BK, D), lambda i: (0, i, 0)),
            pl.BlockSpec((BK, D), lambda i: (i, 0)),
            pl.BlockSpec((BK, 1), lambda i: (i, 0)),
            pl.BlockSpec((1, D), lambda i: (0, 0)),
        ],
        out_specs=[
            pl.BlockSpec((BK, D), lambda i: (i, 0)),
            pl.BlockSpec((1, D), lambda i: (0, 0)),
            pl.BlockSpec((1, D), lambda i: (0, 0)),
        ],
        out_shape=[
            jax.ShapeDtypeStruct((N_PAD, D), jnp.float32),
            jax.ShapeDtypeStruct((1, D), jnp.float32),
            jax.ShapeDtypeStruct((1, D), jnp.float32),
        ],
    )(sp, xw, dis, b)


def _tc_post(h_ref, ssum_ref, ssq_ref, g_ref, be_ref, w_ref, dis_ref,
             xw2_ref, y2_ref):
    mean = ssum_ref[...] * (1.0 / N)
    var = ssq_ref[...] * (1.0 / N) - mean * mean
    inv = lax.rsqrt(var + 1e-5)
    h = (h_ref[...] - mean) * inv * g_ref[...] + be_ref[...]
    h = jnp.maximum(h, 0.0)
    xw2 = jnp.dot(h, w_ref[...], preferred_element_type=jnp.float32)
    xw2_ref[...] = xw2
    y2_ref[...] = xw2 * dis_ref[...]


def _stage_post(h, ssum, ssq, gamma, beta, Wn, dis):
    return pl.pallas_call(
        _tc_post,
        grid=(GRID,),
        in_specs=[
            pl.BlockSpec((BK, D), lambda i: (i, 0)),
            pl.BlockSpec((1, D), lambda i: (0, 0)),
            pl.BlockSpec((1, D), lambda i: (0, 0)),
            pl.BlockSpec((1, D), lambda i: (0, 0)),
            pl.BlockSpec((1, D), lambda i: (0, 0)),
            pl.BlockSpec((D, D), lambda i: (0, 0)),
            pl.BlockSpec((BK, 1), lambda i: (i, 0)),
        ],
        out_specs=[
            pl.BlockSpec((BK, D), lambda i: (i, 0)),
            pl.BlockSpec((BK, D), lambda i: (i, 0)),
        ],
        out_shape=[
            jax.ShapeDtypeStruct((N_PAD, D), jnp.float32),
            jax.ShapeDtypeStruct((N_PAD, D), jnp.float32),
        ],
    )(h, ssum, ssq, gamma, beta, Wn, dis)


def _tc_final(sp_ref, xw_ref, dis_ref, b_ref, out_ref):
    dis = dis_ref[...]
    h = (sp_ref[0] + sp_ref[1]) * dis + xw_ref[...] * (dis * dis) + b_ref[...]
    nrm = jnp.sqrt(jnp.sum(h * h, axis=1, keepdims=True))
    out_ref[...] = h / jnp.maximum(nrm, 1e-12)


def _stage_final(sp, xw, dis, b):
    return pl.pallas_call(
        _tc_final,
        grid=(GRID,),
        in_specs=[
            pl.BlockSpec((NC, BK, D), lambda i: (0, i, 0)),
            pl.BlockSpec((BK, D), lambda i: (i, 0)),
            pl.BlockSpec((BK, 1), lambda i: (i, 0)),
            pl.BlockSpec((1, D), lambda i: (0, 0)),
        ],
        out_specs=pl.BlockSpec((BK, D), lambda i: (i, 0)),
        out_shape=jax.ShapeDtypeStruct((N_PAD, D), jnp.float32),
    )(sp, xw, dis, b)


def kernel(x, edge_index, edge_attr, W1, b1, gamma1, beta1,
           W2, b2, gamma2, beta2, W3, b3):
    r = edge_index[0]
    c = edge_index[1]
    e = r.shape[0]
    pad_e = E_PAD - e
    fill = jnp.arange(pad_e, dtype=jnp.int32)
    r_p = jnp.concatenate([r, fill % N]).reshape(EC, CHUNK)
    c_p = jnp.concatenate([c, fill % N_PAD]).reshape(EC, CHUNK)
    w_p = jnp.concatenate(
        [edge_attr, jnp.zeros((pad_e,), jnp.float32)]).reshape(EC, CHUNK)
    x_pad = jnp.pad(x, ((0, N_PAD - N), (0, 0)))

    degp = _deg_kernel(c_p, w_p).reshape(NC, N_PAD, 1)
    dis, xw1, y1 = _stage_first(x_pad, W1, degp)

    sp1 = _scatter_kernel(y1, r_p, c_p, w_p)
    h1, s1, q1 = _stage_pre(sp1, xw1, dis, b1.reshape(1, D))
    xw2, y2 = _stage_post(h1, s1, q1, gamma1.reshape(1, D),
                          beta1.reshape(1, D), W2, dis)

    sp2 = _scatter_kernel(y2, r_p, c_p, w_p)
    h2, s2, q2 = _stage_pre(sp2, xw2, dis, b2.reshape(1, D))
    xw3, y3 = _stage_post(h2, s2, q2, gamma2.reshape(1, D),
                          beta2.reshape(1, D), W3, dis)

    sp3 = _scatter_kernel(y3, r_p, c_p, w_p)
    out = _stage_final(sp3, xw3, dis, b3.reshape(1, D))
    return out[:N]


# speedup vs baseline: 2.6731x; 1.0207x over previous
import functools

import jax
import jax.numpy as jnp
from jax import lax
from jax.experimental import pallas as pl
from jax.experimental.pallas import tpu as pltpu
from jax.experimental.pallas import tpu_sc as plsc

N = 10000
D = 128
NC = 2
NS = 16
NW = NC * NS
LANES = 16
N_PAD = 10240
RPT = N_PAD // NS
CHUNK = 112
SUP = 8
E_PAD = 344064
EC = E_PAD // CHUNK
CPT = EC // NW
NSUP = CPT // SUP

BK = 1024
GRID = N_PAD // BK

_mesh = plsc.VectorSubcoreMesh(
    core_axis_name="c", subcore_axis_name="s", num_cores=NC, num_subcores=NS
)


@functools.partial(
    pl.kernel,
    out_type=jax.ShapeDtypeStruct((NC, N_PAD), jnp.float32),
    mesh=_mesh,
    scratch_types=[
        pltpu.VMEM((CPT, CHUNK), jnp.int32),
        pltpu.VMEM((CPT, CHUNK), jnp.float32),
        pltpu.VMEM((RPT,), jnp.float32),
        pltpu.VMEM_SHARED((N_PAD,), jnp.float32),
        pltpu.SemaphoreType.DMA,
    ],
)
def _deg_kernel(c_hbm, w_hbm, degp_hbm, cbuf, wbuf, zbuf, degs, ssem):
    cid = lax.axis_index("c")
    sid = lax.axis_index("s")
    wid = cid * NS + sid
    base = wid * CPT

    pltpu.sync_copy(c_hbm.at[pl.ds(base, CPT)], cbuf)
    pltpu.sync_copy(w_hbm.at[pl.ds(base, CPT)], wbuf)

    def _z(i, carry):
        zbuf[pl.ds(i * LANES, LANES)] = jnp.zeros((LANES,), jnp.float32)
        return carry

    lax.fori_loop(0, RPT // LANES, _z, None)
    pltpu.sync_copy(zbuf, degs.at[pl.ds(sid * RPT, RPT)])
    plsc.subcore_barrier()

    GRP = 16

    def _fire(j, carry):
        pltpu.async_copy(wbuf.at[j], degs.at[cbuf.at[j]], ssem, add=True)
        return carry

    def _drain(j, carry):
        pltpu.make_async_copy(wbuf.at[j], degs.at[cbuf.at[j]], ssem).wait()
        return carry

    for g in range(CPT // GRP):
        lax.fori_loop(g * GRP, (g + 1) * GRP, _fire, None)
        lax.fori_loop(g * GRP, (g + 1) * GRP, _drain, None)
    plsc.subcore_barrier()

    @pl.when(sid == 0)
    def _():
        pltpu.sync_copy(degs, degp_hbm.at[cid])


@functools.partial(
    pl.kernel,
    out_type=jax.ShapeDtypeStruct((NC, N_PAD, D), jnp.float32),
    mesh=_mesh,
    scratch_types=[
        pltpu.VMEM((2, SUP, CHUNK), jnp.int32),
        pltpu.VMEM((2, SUP, CHUNK), jnp.int32),
        pltpu.VMEM((2, SUP, CHUNK), jnp.float32),
        pltpu.VMEM((2, CHUNK, D), jnp.float32),
        pltpu.VMEM((CHUNK, D), jnp.float32),
        pltpu.VMEM_SHARED((N_PAD, D), jnp.float32),
        pltpu.SemaphoreType.DMA,
        pltpu.SemaphoreType.DMA,
        pltpu.SemaphoreType.DMA,
    ],
)
def _scatter_kernel(y_hbm, r_hbm, c_hbm, w_hbm, sp_hbm,
                    rbuf, cbuf, wbuf, gbuf, sbuf, acc, gsem, ssem, isem):
    cid = lax.axis_index("c")
    sid = lax.axis_index("s")
    wid = cid * NS + sid
    tb = wid * CPT

    pltpu.sync_copy(r_hbm.at[pl.ds(tb, SUP)], rbuf.at[0])
    pltpu.sync_copy(c_hbm.at[pl.ds(tb, SUP)], cbuf.at[0])
    pltpu.sync_copy(w_hbm.at[pl.ds(tb, SUP)], wbuf.at[0])
    pltpu.async_copy(y_hbm.at[rbuf.at[0, 0]], gbuf.at[0], gsem)
    pltpu.async_copy(y_hbm.at[rbuf.at[0, 1]], gbuf.at[1], gsem)

    def _zr(i, carry):
        for d in range(D // LANES):
            sbuf[i, pl.ds(d * LANES, LANES)] = jnp.zeros((LANES,), jnp.float32)
        return carry

    lax.fori_loop(0, CHUNK, _zr, None)
    NZ = RPT // CHUNK
    TAIL = RPT - NZ * CHUNK
    for k in range(NZ):
        pltpu.sync_copy(sbuf, acc.at[pl.ds(sid * RPT + k * CHUNK, CHUNK)])
    if TAIL:
        pltpu.sync_copy(sbuf.at[pl.ds(0, TAIL)],
                        acc.at[pl.ds(sid * RPT + NZ * CHUNK, TAIL)])
    plsc.subcore_barrier()

    def _sup(s, carry):
        p = lax.rem(s, 2)
        pn = 1 - p
        nrow0 = tb + (s + 1) * SUP
        for j in range(SUP):
            b = j % 2
            pltpu.make_async_copy(
                y_hbm.at[rbuf.at[p, j]], gbuf.at[b], gsem).wait()

            if j >= 1:
                pltpu.make_async_copy(
                    sbuf, acc.at[cbuf.at[p, j - 1]], ssem).wait()
            else:
                @pl.when(s >= 1)
                def _():
                    pltpu.make_async_copy(
                        sbuf, acc.at[cbuf.at[pn, SUP - 1]], ssem).wait()

            if j == 2:
                @pl.when(s < NSUP - 1)
                def _():
                    pltpu.async_copy(
                        r_hbm.at[pl.ds(nrow0, SUP)], rbuf.at[pn], isem)
                    pltpu.async_copy(
                        c_hbm.at[pl.ds(nrow0, SUP)], cbuf.at[pn], isem)
                    pltpu.async_copy(
                        w_hbm.at[pl.ds(nrow0, SUP)], wbuf.at[pn], isem)

            @plsc.parallel_loop(0, CHUNK // LANES, unroll=2)
            def _mul(gg):
                wv = wbuf[p, j, pl.ds(gg * LANES, LANES)]
                for l in range(LANES):
                    bv = jnp.full((LANES,), wv[l], jnp.float32)
                    for d in range(D // LANES):
                        sl = pl.ds(d * LANES, LANES)
                        sbuf[gg * LANES + l, sl] = (
                            gbuf[b, gg * LANES + l, sl] * bv)

            pltpu.async_copy(sbuf, acc.at[cbuf.at[p, j]], ssem, add=True)

            if j == 5:
                @pl.when(s < NSUP - 1)
                def _():
                    pltpu.make_async_copy(
                        r_hbm.at[pl.ds(nrow0, SUP)], rbuf.at[pn], isem).wait()
                    pltpu.make_async_copy(
                        c_hbm.at[pl.ds(nrow0, SUP)], cbuf.at[pn], isem).wait()
                    pltpu.make_async_copy(
                        w_hbm.at[pl.ds(nrow0, SUP)], wbuf.at[pn], isem).wait()

            if j < SUP - 2:
                pltpu.async_copy(y_hbm.at[rbuf.at[p, j + 2]], gbuf.at[b], gsem)
            else:
                @pl.when(s < NSUP - 1)
                def _():
                    pltpu.async_copy(y_hbm.at[rbuf.at[pn, j - (SUP - 2)]],
                                     gbuf.at[b], gsem)
        return carry

    lax.fori_loop(0, NSUP, _sup, None)
    pltpu.make_async_copy(
        sbuf, acc.at[cbuf.at[(NSUP - 1) % 2, SUP - 1]], ssem).wait()
    plsc.subcore_barrier()

    NZ2 = RPT // CHUNK
    for k in range(NZ2):
        r0 = sid * RPT + k * CHUNK
        pltpu.sync_copy(acc.at[pl.ds(r0, CHUNK)],
                        sp_hbm.at[cid, pl.ds(r0, CHUNK)])
    if RPT - NZ2 * CHUNK:
        r0 = sid * RPT + NZ2 * CHUNK
        t = RPT - NZ2 * CHUNK
        pltpu.sync_copy(acc.at[pl.ds(r0, t)], sp_hbm.at[cid, pl.ds(r0, t)])


def _tc_first(x_ref, w_ref, degp_ref, dis_ref, xw_ref, y_ref):
    deg = 1.0 + degp_ref[0] + degp_ref[1]
    dis = lax.rsqrt(deg)
    xw = jnp.dot(x_ref[...], w_ref[...], preferred_element_type=jnp.float32)
    dis_ref[...] = dis
    xw_ref[...] = xw
    y_ref[...] = xw * dis


def _stage_first(x_pad, W1, degp):
    return pl.pallas_call(
        _tc_first,
        grid=(GRID,),
        in_specs=[
            pl.BlockSpec((BK, D), lambda i: (i, 0)),
            pl.BlockSpec((D, D), lambda i: (0, 0)),
            pl.BlockSpec((NC, BK, 1), lambda i: (0, i, 0)),
        ],
        out_specs=[
            pl.BlockSpec((BK, 1), lambda i: (i, 0)),
            pl.BlockSpec((BK, D), lambda i: (i, 0)),
            pl.BlockSpec((BK, D), lambda i: (i, 0)),
        ],
        out_shape=[
            jax.ShapeDtypeStruct((N_PAD, 1), jnp.float32),
            jax.ShapeDtypeStruct((N_PAD, D), jnp.float32),
            jax.ShapeDtypeStruct((N_PAD, D), jnp.float32),
        ],
    )(x_pad, W1, degp)


def _tc_pre(sp_ref, xw_ref, dis_ref, b_ref, h_ref, ssum_ref, ssq_ref):
    i = pl.program_id(0)
    dis = dis_ref[...]
    h = (sp_ref[0] + sp_ref[1]) * dis + xw_ref[...] * (dis * dis) + b_ref[...]
    ridx = lax.broadcasted_iota(jnp.int32, (BK, 1), 0) + i * BK
    h = h * (ridx < N).astype(jnp.float32)
    h_ref[...] = h

    @pl.when(i == 0)
    def _():
        ssum_ref[...] = jnp.zeros_like(ssum_ref)
        ssq_ref[...] = jnp.zeros_like(ssq_ref)

    ssum_ref[...] += jnp.sum(h, axis=0, keepdims=True)
    ssq_ref[...] += jnp.sum(h * h, axis=0, keepdims=True)


def _stage_pre(sp, xw, dis, b):
    return pl.pallas_call(
        _tc_pre,
        grid=(GRID,),
        in_specs=[
            pl.BlockSpec((NC, BK, D), lambda i: (0, i, 0)),
            pl.BlockSpec((BK, D), lambda i: (i, 0)),
            pl.BlockSpec((BK, 1), lambda i: (i, 0)),
            pl.BlockSpec((1, D), lambda i: (0, 0)),
        ],
        out_specs=[
            pl.BlockSpec((BK, D), lambda i: (i, 0)),
            pl.BlockSpec((1, D), lambda i: (0, 0)),
            pl.BlockSpec((1, D), lambda i: (0, 0)),
        ],
        out_shape=[
            jax.ShapeDtypeStruct((N_PAD, D), jnp.float32),
            jax.ShapeDtypeStruct((1, D), jnp.float32),
            jax.ShapeDtypeStruct((1, D), jnp.float32),
        ],
    )(sp, xw, dis, b)


def _tc_post(h_ref, ssum_ref, ssq_ref, g_ref, be_ref, w_ref, dis_ref,
             xw2_ref, y2_ref):
    mean = ssum_ref[...] * (1.0 / N)
    var = ssq_ref[...] * (1.0 / N) - mean * mean
    inv = lax.rsqrt(var + 1e-5)
    h = (h_ref[...] - mean) * inv * g_ref[...] + be_ref[...]
    h = jnp.maximum(h, 0.0)
    xw2 = jnp.dot(h, w_ref[...], preferred_element_type=jnp.float32)
    xw2_ref[...] = xw2
    y2_ref[...] = xw2 * dis_ref[...]


def _stage_post(h, ssum, ssq, gamma, beta, Wn, dis):
    return pl.pallas_call(
        _tc_post,
        grid=(GRID,),
        in_specs=[
            pl.BlockSpec((BK, D), lambda i: (i, 0)),
            pl.BlockSpec((1, D), lambda i: (0, 0)),
            pl.BlockSpec((1, D), lambda i: (0, 0)),
            pl.BlockSpec((1, D), lambda i: (0, 0)),
            pl.BlockSpec((1, D), lambda i: (0, 0)),
            pl.BlockSpec((D, D), lambda i: (0, 0)),
            pl.BlockSpec((BK, 1), lambda i: (i, 0)),
        ],
        out_specs=[
            pl.BlockSpec((BK, D), lambda i: (i, 0)),
            pl.BlockSpec((BK, D), lambda i: (i, 0)),
        ],
        out_shape=[
            jax.ShapeDtypeStruct((N_PAD, D), jnp.float32),
            jax.ShapeDtypeStruct((N_PAD, D), jnp.float32),
        ],
    )(h, ssum, ssq, gamma, beta, Wn, dis)


def _tc_final(sp_ref, xw_ref, dis_ref, b_ref, out_ref):
    dis = dis_ref[...]
    h = (sp_ref[0] + sp_ref[1]) * dis + xw_ref[...] * (dis * dis) + b_ref[...]
    nrm = jnp.sqrt(jnp.sum(h * h, axis=1, keepdims=True))
    out_ref[...] = h / jnp.maximum(nrm, 1e-12)


def _stage_final(sp, xw, dis, b):
    return pl.pallas_call(
        _tc_final,
        grid=(GRID,),
        in_specs=[
            pl.BlockSpec((NC, BK, D), lambda i: (0, i, 0)),
            pl.BlockSpec((BK, D), lambda i: (i, 0)),
            pl.BlockSpec((BK, 1), lambda i: (i, 0)),
            pl.BlockSpec((1, D), lambda i: (0, 0)),
        ],
        out_specs=pl.BlockSpec((BK, D), lambda i: (i, 0)),
        out_shape=jax.ShapeDtypeStruct((N_PAD, D), jnp.float32),
    )(sp, xw, dis, b)


def kernel(x, edge_index, edge_attr, W1, b1, gamma1, beta1,
           W2, b2, gamma2, beta2, W3, b3):
    r = edge_index[0]
    c = edge_index[1]
    e = r.shape[0]
    pad_e = E_PAD - e
    fill = jnp.arange(pad_e, dtype=jnp.int32)
    r_p = jnp.concatenate([r, fill % N]).reshape(EC, CHUNK)
    c_p = jnp.concatenate([c, fill % N_PAD]).reshape(EC, CHUNK)
    w_p = jnp.concatenate(
        [edge_attr, jnp.zeros((pad_e,), jnp.float32)]).reshape(EC, CHUNK)
    x_pad = jnp.pad(x, ((0, N_PAD - N), (0, 0)))

    degp = _deg_kernel(c_p, w_p).reshape(NC, N_PAD, 1)
    dis, xw1, y1 = _stage_first(x_pad, W1, degp)

    sp1 = _scatter_kernel(y1, r_p, c_p, w_p)
    h1, s1, q1 = _stage_pre(sp1, xw1, dis, b1.reshape(1, D))
    xw2, y2 = _stage_post(h1, s1, q1, gamma1.reshape(1, D),
                          beta1.reshape(1, D), W2, dis)

    sp2 = _scatter_kernel(y2, r_p, c_p, w_p)
    h2, s2, q2 = _stage_pre(sp2, xw2, dis, b2.reshape(1, D))
    xw3, y3 = _stage_post(h2, s2, q2, gamma2.reshape(1, D),
                          beta2.reshape(1, D), W3, dis)

    sp3 = _scatter_kernel(y3, r_p, c_p, w_p)
    out = _stage_final(sp3, xw3, dis, b3.reshape(1, D))
    return out[:N]


# speedup vs baseline: 2.6805x; 1.0028x over previous
import functools

import jax
import jax.numpy as jnp
from jax import lax
from jax.experimental import pallas as pl
from jax.experimental.pallas import tpu as pltpu
from jax.experimental.pallas import tpu_sc as plsc

N = 10000
D = 128
NC = 2
NS = 16
NW = NC * NS
LANES = 16
N_PAD = 10240
RPT = N_PAD // NS
CHUNK = 112
SUP = 8
E_PAD = 344064
EC = E_PAD // CHUNK
CPT = EC // NW
NSUP = CPT // SUP

BK = 1024
GRID = N_PAD // BK

_mesh = plsc.VectorSubcoreMesh(
    core_axis_name="c", subcore_axis_name="s", num_cores=NC, num_subcores=NS
)


@functools.partial(
    pl.kernel,
    out_type=jax.ShapeDtypeStruct((NC, N_PAD), jnp.float32),
    mesh=_mesh,
    scratch_types=[
        pltpu.VMEM((CPT, CHUNK), jnp.int32),
        pltpu.VMEM((CPT, CHUNK), jnp.float32),
        pltpu.VMEM((RPT,), jnp.float32),
        pltpu.VMEM_SHARED((N_PAD,), jnp.float32),
        pltpu.SemaphoreType.DMA,
    ],
)
def _deg_kernel(c_hbm, w_hbm, degp_hbm, cbuf, wbuf, zbuf, degs, ssem):
    cid = lax.axis_index("c")
    sid = lax.axis_index("s")
    wid = cid * NS + sid
    base = wid * CPT

    pltpu.sync_copy(c_hbm.at[pl.ds(base, CPT)], cbuf)
    pltpu.sync_copy(w_hbm.at[pl.ds(base, CPT)], wbuf)

    def _z(i, carry):
        zbuf[pl.ds(i * LANES, LANES)] = jnp.zeros((LANES,), jnp.float32)
        return carry

    lax.fori_loop(0, RPT // LANES, _z, None)
    pltpu.sync_copy(zbuf, degs.at[pl.ds(sid * RPT, RPT)])
    plsc.subcore_barrier()

    GRP = 16

    def _fire(j, carry):
        pltpu.async_copy(wbuf.at[j], degs.at[cbuf.at[j]], ssem, add=True)
        return carry

    def _drain(j, carry):
        pltpu.make_async_copy(wbuf.at[j], degs.at[cbuf.at[j]], ssem).wait()
        return carry

    for g in range(CPT // GRP):
        lax.fori_loop(g * GRP, (g + 1) * GRP, _fire, None)
        lax.fori_loop(g * GRP, (g + 1) * GRP, _drain, None)
    plsc.subcore_barrier()

    @pl.when(sid == 0)
    def _():
        pltpu.sync_copy(degs, degp_hbm.at[cid])


@functools.partial(
    pl.kernel,
    out_type=jax.ShapeDtypeStruct((NC, N_PAD, D), jnp.float32),
    mesh=_mesh,
    scratch_types=[
        pltpu.VMEM((2, SUP, CHUNK), jnp.int32),
        pltpu.VMEM((2, SUP, CHUNK), jnp.int32),
        pltpu.VMEM((2, SUP, CHUNK), jnp.float32),
        pltpu.VMEM((2, CHUNK, D), jnp.float32),
        pltpu.VMEM((CHUNK, D), jnp.float32),
        pltpu.VMEM_SHARED((N_PAD, D), jnp.float32),
        pltpu.SemaphoreType.DMA,
        pltpu.SemaphoreType.DMA,
        pltpu.SemaphoreType.DMA,
    ],
)
def _scatter_kernel(y_hbm, r_hbm, c_hbm, w_hbm, sp_hbm,
                    rbuf, cbuf, wbuf, gbuf, sbuf, acc, gsem, ssem, isem):
    cid = lax.axis_index("c")
    sid = lax.axis_index("s")
    wid = cid * NS + sid
    tb = wid * CPT

    pltpu.sync_copy(r_hbm.at[pl.ds(tb, SUP)], rbuf.at[0])
    pltpu.sync_copy(c_hbm.at[pl.ds(tb, SUP)], cbuf.at[0])
    pltpu.sync_copy(w_hbm.at[pl.ds(tb, SUP)], wbuf.at[0])
    pltpu.async_copy(y_hbm.at[rbuf.at[0, 0]], gbuf.at[0], gsem)
    pltpu.async_copy(y_hbm.at[rbuf.at[0, 1]], gbuf.at[1], gsem)

    def _zr(i, carry):
        for d in range(D // LANES):
            sbuf[i, pl.ds(d * LANES, LANES)] = jnp.zeros((LANES,), jnp.float32)
        return carry

    lax.fori_loop(0, CHUNK, _zr, None)
    NZ = RPT // CHUNK
    TAIL = RPT - NZ * CHUNK
    for k in range(NZ):
        pltpu.sync_copy(sbuf, acc.at[pl.ds(sid * RPT + k * CHUNK, CHUNK)])
    if TAIL:
        pltpu.sync_copy(sbuf.at[pl.ds(0, TAIL)],
                        acc.at[pl.ds(sid * RPT + NZ * CHUNK, TAIL)])
    plsc.subcore_barrier()

    def _sup(s, carry):
        p = lax.rem(s, 2)
        pn = 1 - p
        nrow0 = tb + (s + 1) * SUP
        for j in range(SUP):
            b = j % 2
            pltpu.make_async_copy(
                y_hbm.at[rbuf.at[p, j]], gbuf.at[b], gsem).wait()

            if j >= 1:
                pltpu.make_async_copy(
                    sbuf, acc.at[cbuf.at[p, j - 1]], ssem).wait()
            else:
                @pl.when(s >= 1)
                def _():
                    pltpu.make_async_copy(
                        sbuf, acc.at[cbuf.at[pn, SUP - 1]], ssem).wait()

            if j == 2:
                @pl.when(s < NSUP - 1)
                def _():
                    pltpu.async_copy(
                        r_hbm.at[pl.ds(nrow0, SUP)], rbuf.at[pn], isem)
                    pltpu.async_copy(
                        c_hbm.at[pl.ds(nrow0, SUP)], cbuf.at[pn], isem)
                    pltpu.async_copy(
                        w_hbm.at[pl.ds(nrow0, SUP)], wbuf.at[pn], isem)

            @plsc.parallel_loop(0, CHUNK // LANES, unroll=2)
            def _mul(gg):
                wv = wbuf[p, j, pl.ds(gg * LANES, LANES)]
                for l in range(LANES):
                    bv = jnp.full((LANES,), wv[l], jnp.float32)
                    for d in range(D // LANES):
                        sl = pl.ds(d * LANES, LANES)
                        sbuf[gg * LANES + l, sl] = (
                            gbuf[b, gg * LANES + l, sl] * bv)

            pltpu.async_copy(sbuf, acc.at[cbuf.at[p, j]], ssem, add=True)

            if j == 5:
                @pl.when(s < NSUP - 1)
                def _():
                    pltpu.make_async_copy(
                        r_hbm.at[pl.ds(nrow0, SUP)], rbuf.at[pn], isem).wait()
                    pltpu.make_async_copy(
                        c_hbm.at[pl.ds(nrow0, SUP)], cbuf.at[pn], isem).wait()
                    pltpu.make_async_copy(
                        w_hbm.at[pl.ds(nrow0, SUP)], wbuf.at[pn], isem).wait()

            if j < SUP - 2:
                pltpu.async_copy(y_hbm.at[rbuf.at[p, j + 2]], gbuf.at[b], gsem)
            else:
                @pl.when(s < NSUP - 1)
                def _():
                    pltpu.async_copy(y_hbm.at[rbuf.at[pn, j - (SUP - 2)]],
                                     gbuf.at[b], gsem)
        return carry

    lax.fori_loop(0, NSUP, _sup, None)
    pltpu.make_async_copy(
        sbuf, acc.at[cbuf.at[(NSUP - 1) % 2, SUP - 1]], ssem).wait()
    plsc.subcore_barrier()

    NZ2 = RPT // CHUNK
    for k in range(NZ2):
        r0 = sid * RPT + k * CHUNK
        pltpu.sync_copy(acc.at[pl.ds(r0, CHUNK)],
                        sp_hbm.at[cid, pl.ds(r0, CHUNK)])
    if RPT - NZ2 * CHUNK:
        r0 = sid * RPT + NZ2 * CHUNK
        t = RPT - NZ2 * CHUNK
        pltpu.sync_copy(acc.at[pl.ds(r0, t)], sp_hbm.at[cid, pl.ds(r0, t)])


def _tc_first(x_ref, w_ref, degp_ref, dis_ref, xw_ref, y_ref):
    deg = 1.0 + degp_ref[0] + degp_ref[1]
    dis = lax.rsqrt(deg)
    xw = jnp.dot(x_ref[...], w_ref[...], preferred_element_type=jnp.float32)
    dis_ref[...] = dis
    xw_ref[...] = xw
    y_ref[...] = xw * dis


def _stage_first(x_pad, W1, degp):
    return pl.pallas_call(
        _tc_first,
        grid=(GRID,),
        in_specs=[
            pl.BlockSpec((BK, D), lambda i: (i, 0)),
            pl.BlockSpec((D, D), lambda i: (0, 0)),
            pl.BlockSpec((NC, BK, 1), lambda i: (0, i, 0)),
        ],
        out_specs=[
            pl.BlockSpec((BK, 1), lambda i: (i, 0)),
            pl.BlockSpec((BK, D), lambda i: (i, 0)),
            pl.BlockSpec((BK, D), lambda i: (i, 0)),
        ],
        out_shape=[
            jax.ShapeDtypeStruct((N_PAD, 1), jnp.float32),
            jax.ShapeDtypeStruct((N_PAD, D), jnp.float32),
            jax.ShapeDtypeStruct((N_PAD, D), jnp.float32),
        ],
    )(x_pad, W1, degp)


def _tc_mid(sp_ref, xw_ref, dis_ref, b_ref, g_ref, be_ref, w_ref,
            xw2_ref, y2_ref, hbuf, stat):
    ph = pl.program_id(0)
    i = pl.program_id(1)
    dis = dis_ref[...]

    @pl.when(ph == 0)
    def _():
        h = ((sp_ref[0] + sp_ref[1]) * dis + xw_ref[...] * (dis * dis)
             + b_ref[...])
        ridx = lax.broadcasted_iota(jnp.int32, (BK, 1), 0) + i * BK
        h = h * (ridx < N).astype(jnp.float32)
        hbuf[pl.ds(i * BK, BK), :] = h

        @pl.when(i == 0)
        def _():
            stat[...] = jnp.zeros_like(stat)

        stat[0:1, :] += jnp.sum(h, axis=0, keepdims=True)
        stat[1:2, :] += jnp.sum(h * h, axis=0, keepdims=True)

    @pl.when(ph == 1)
    def _():
        mean = stat[0:1, :] * (1.0 / N)
        var = stat[1:2, :] * (1.0 / N) - mean * mean
        inv = lax.rsqrt(var + 1e-5)
        h2 = (hbuf[pl.ds(i * BK, BK), :] - mean) * inv * g_ref[...] + be_ref[...]
        h2 = jnp.maximum(h2, 0.0)
        xw2 = jnp.dot(h2, w_ref[...], preferred_element_type=jnp.float32)
        xw2_ref[...] = xw2
        y2_ref[...] = xw2 * dis


def _stage_mid(sp, xw, dis, b, gamma, beta, Wn):
    return pl.pallas_call(
        _tc_mid,
        grid=(2, GRID),
        in_specs=[
            pl.BlockSpec((NC, BK, D), lambda ph, i: (0, i * (1 - ph), 0)),
            pl.BlockSpec((BK, D), lambda ph, i: (i * (1 - ph), 0)),
            pl.BlockSpec((BK, 1), lambda ph, i: (i, 0)),
            pl.BlockSpec((1, D), lambda ph, i: (0, 0)),
            pl.BlockSpec((1, D), lambda ph, i: (0, 0)),
            pl.BlockSpec((1, D), lambda ph, i: (0, 0)),
            pl.BlockSpec((D, D), lambda ph, i: (0, 0)),
        ],
        out_specs=[
            pl.BlockSpec((BK, D), lambda ph, i: (i, 0)),
            pl.BlockSpec((BK, D), lambda ph, i: (i, 0)),
        ],
        out_shape=[
            jax.ShapeDtypeStruct((N_PAD, D), jnp.float32),
            jax.ShapeDtypeStruct((N_PAD, D), jnp.float32),
        ],
        scratch_shapes=[
            pltpu.VMEM((N_PAD, D), jnp.float32),
            pltpu.VMEM((2, D), jnp.float32),
        ],
    )(sp, xw, dis, b, gamma, beta, Wn)


def _tc_final(sp_ref, xw_ref, dis_ref, b_ref, out_ref):
    dis = dis_ref[...]
    h = (sp_ref[0] + sp_ref[1]) * dis + xw_ref[...] * (dis * dis) + b_ref[...]
    nrm = jnp.sqrt(jnp.sum(h * h, axis=1, keepdims=True))
    out_ref[...] = h / jnp.maximum(nrm, 1e-12)


def _stage_final(sp, xw, dis, b):
    return pl.pallas_call(
        _tc_final,
        grid=(GRID,),
        in_specs=[
            pl.BlockSpec((NC, BK, D), lambda i: (0, i, 0)),
            pl.BlockSpec((BK, D), lambda i: (i, 0)),
            pl.BlockSpec((BK, 1), lambda i: (i, 0)),
            pl.BlockSpec((1, D), lambda i: (0, 0)),
        ],
        out_specs=pl.BlockSpec((BK, D), lambda i: (i, 0)),
        out_shape=jax.ShapeDtypeStruct((N_PAD, D), jnp.float32),
    )(sp, xw, dis, b)


def kernel(x, edge_index, edge_attr, W1, b1, gamma1, beta1,
           W2, b2, gamma2, beta2, W3, b3):
    r = edge_index[0]
    c = edge_index[1]
    e = r.shape[0]
    pad_e = E_PAD - e
    fill = jnp.arange(pad_e, dtype=jnp.int32)
    r_p = jnp.concatenate([r, fill % N]).reshape(EC, CHUNK)
    c_p = jnp.concatenate([c, fill % N_PAD]).reshape(EC, CHUNK)
    w_p = jnp.concatenate(
        [edge_attr, jnp.zeros((pad_e,), jnp.float32)]).reshape(EC, CHUNK)
    x_pad = jnp.pad(x, ((0, N_PAD - N), (0, 0)))

    degp = _deg_kernel(c_p, w_p).reshape(NC, N_PAD, 1)
    dis, xw1, y1 = _stage_first(x_pad, W1, degp)

    sp1 = _scatter_kernel(y1, r_p, c_p, w_p)
    xw2, y2 = _stage_mid(sp1, xw1, dis, b1.reshape(1, D),
                         gamma1.reshape(1, D), beta1.reshape(1, D), W2)

    sp2 = _scatter_kernel(y2, r_p, c_p, w_p)
    xw3, y3 = _stage_mid(sp2, xw2, dis, b2.reshape(1, D),
                         gamma2.reshape(1, D), beta2.reshape(1, D), W3)

    sp3 = _scatter_kernel(y3, r_p, c_p, w_p)
    out = _stage_final(sp3, xw3, dis, b3.reshape(1, D))
    return out[:N]
